# baseline (speedup 1.0000x reference)
import functools
import jax
import jax.numpy as jnp
from jax import lax
from jax.experimental import pallas as pl
from jax.experimental.pallas import tpu as pltpu
from jax.experimental.pallas import tpu_sc as plsc

N = 8192
D = 64
HHR = 6
TOPK = 10
NEG = -3.0e38

BI = 256
RB = 64
BA = 512
LANE = 128

B = 1024
L = 16
NW = 32
IPW = (B // NW) * L
BT = 128


def _hidden_norm_body(x_ref, vs_ref, o_ref):
    h = x_ref[...]
    for i in range(HHR):
        v = vs_ref[i:i + 1, :]
        vdot = jnp.sum(v * v)
        w = lax.dot_general(h, v, (((1,), (1,)), ((), ())),
                            preferred_element_type=jnp.float32)
        h = h - lax.dot_general(w, v, (((1,), (0,)), ((), ())),
                                preferred_element_type=jnp.float32) / vdot
    norm = jnp.sqrt(jnp.sum(h * h, axis=1, keepdims=True))
    o_ref[...] = h / jnp.maximum(norm, 1e-12)


def _hidden_norm(x, vs):
    return pl.pallas_call(
        _hidden_norm_body,
        grid=(N // BA,),
        in_specs=[
            pl.BlockSpec((BA, D), lambda i: (i, 0)),
            pl.BlockSpec((HHR, D), lambda i: (0, 0)),
        ],
        out_specs=pl.BlockSpec((BA, D), lambda i: (i, 0)),
        out_shape=jax.ShapeDtypeStruct((N, D), jnp.float32),
    )(x, vs)


def _topk_table_body(a_ref, b_ref, o_ref):
    s = lax.dot_general(a_ref[...], b_ref[...], (((1,), (1,)), ((), ())),
                        preferred_element_type=jnp.float32)
    sh = s.astype(jnp.bfloat16)
    parts = []
    for rb in range(BI // RB):
        t = [jnp.full((RB, LANE), NEG, dtype=jnp.bfloat16)
             for _ in range(TOPK)]
        for c in range(N // LANE):
            new = sh[rb * RB:(rb + 1) * RB, c * LANE:(c + 1) * LANE]
            for r in range(TOPK):
                hi = jnp.maximum(t[r], new)
                new = jnp.minimum(t[r], new)
                t[r] = hi
        lists = t
        w = LANE
        while w > 1:
            h = w // 2
            a = [x[:, :h] for x in lists]
            b = [x[:, h:w] for x in lists]
            nxt = []
            for k in range(TOPK):
                m = jnp.maximum(a[k], b[k])
                for i in range(k):
                    m = jnp.maximum(m, jnp.minimum(a[i], b[k - 1 - i]))
                nxt.append(m)
            lists = nxt
            w = h
        total = lists[0].astype(jnp.float32)
        for k in range(1, TOPK):
            total = total + lists[k].astype(jnp.float32)
        parts.append(total)
    total = jnp.concatenate(parts, axis=0)
    o_ref[...] = jnp.broadcast_to(total * (1.0 / TOPK), (BI, 16))


def _topk_table(a, b):
    return pl.pallas_call(
        _topk_table_body,
        grid=(N // BI,),
        in_specs=[
            pl.BlockSpec((BI, D), lambda i: (i, 0)),
            pl.BlockSpec((N, D), lambda i: (0, 0)),
        ],
        out_specs=pl.BlockSpec((BI, 16), lambda i: (i, 0)),
        out_shape=jax.ShapeDtypeStruct((N, 16), jnp.float32),
    )(a, b)


def _make_sc_gather(width):
    mesh = plsc.VectorSubcoreMesh(core_axis_name="c", subcore_axis_name="s",
                                  num_cores=2, num_subcores=16)

    @functools.partial(
        pl.kernel,
        out_type=[jax.ShapeDtypeStruct((NW, IPW, width), jnp.float32)] * 2,
        mesh=mesh,
        scratch_types=[
            pltpu.VMEM((IPW // 128, 128), jnp.int32),
            pltpu.VMEM((IPW // 128, 128), jnp.int32),
            pltpu.VMEM((IPW, width), jnp.float32),
            pltpu.SemaphoreType.DMA,
        ],
        compiler_params=pltpu.CompilerParams(use_tc_tiling_on_sc=False),
    )
    def sc_gather(stab_h, ttab_h, si_h, ti_h, so_h, to_h,
                  si_v, ti_v, rows_v, sem):
        wid = lax.axis_index("s") * 2 + lax.axis_index("c")
        pltpu.sync_copy(si_h.at[wid], si_v)
        pltpu.sync_copy(ti_h.at[wid], ti_v)
        for j in range(IPW // 128):
            pltpu.async_copy(stab_h.at[si_v.at[j]],
                             rows_v.at[pl.ds(j * 128, 128)], sem).wait()
        pltpu.sync_copy(rows_v, so_h.at[wid])
        for j in range(IPW // 128):
            pltpu.async_copy(ttab_h.at[ti_v.at[j]],
                             rows_v.at[pl.ds(j * 128, 128)], sem).wait()
        pltpu.sync_copy(rows_v, to_h.at[wid])

    return sc_gather


@functools.lru_cache(maxsize=1)
def _sc_gather_rows():
    return _make_sc_gather(D)


@functools.lru_cache(maxsize=1)
def _sc_gather_scores():
    return _make_sc_gather(16)


def _tail_body(sg_ref, tg_ref, rtg_ref, rsg_ref, o1_ref, o2_ref):
    sl = sg_ref[...].reshape(BT, L, D)
    tl = tg_ref[...].reshape(BT, L, D)
    rtv = rtg_ref[...].reshape(BT, L, 16)[:, :, 0]
    rsv = rsg_ref[...].reshape(BT, L, 16)[:, :, 0]
    sr0 = sl[:, 0:1, :]
    tr0 = tl[:, 0:1, :]
    s2t = jnp.sum(sr0 * tl, axis=2)
    t2s = jnp.sum(tr0 * sl, axis=2)
    o1_ref[...] = 2.0 * s2t - rtv[:, 0:1] - rsv
    o2_ref[...] = 2.0 * t2s - rsv[:, 0:1] - rtv


def _tail(sg, tg, rtg, rsg):
    return pl.pallas_call(
        _tail_body,
        grid=(B // BT,),
        in_specs=[
            pl.BlockSpec((BT * L, D), lambda i: (i, 0)),
            pl.BlockSpec((BT * L, D), lambda i: (i, 0)),
            pl.BlockSpec((BT * L, 16), lambda i: (i, 0)),
            pl.BlockSpec((BT * L, 16), lambda i: (i, 0)),
        ],
        out_specs=[
            pl.BlockSpec((BT, L), lambda i: (i, 0)),
            pl.BlockSpec((BT, L), lambda i: (i, 0)),
        ],
        out_shape=[jax.ShapeDtypeStruct((B, L), jnp.float32)] * 2,
    )(sg, tg, rtg, rsg)


@jax.jit
def kernel(node_feat_src, node_feat_tgt, srcs_index, tgts_index, src_vs, tgt_vs):
    src_vs2 = src_vs.reshape(HHR, D)
    tgt_vs2 = tgt_vs.reshape(HHR, D)
    src_n = _hidden_norm(node_feat_src, src_vs2)
    tgt_n = _hidden_norm(node_feat_tgt, tgt_vs2)

    si3 = srcs_index.astype(jnp.int32).reshape(NW, IPW // 128, 128)
    ti3 = tgts_index.astype(jnp.int32).reshape(NW, IPW // 128, 128)

    sg, tg = _sc_gather_rows()(src_n, tgt_n, si3, ti3)

    rt16 = _topk_table(src_n, tgt_n)
    rs16 = _topk_table(tgt_n, src_n)

    rtg, rsg = _sc_gather_scores()(rt16, rs16, si3, ti3)

    o1, o2 = _tail(sg.reshape(B * L, D), tg.reshape(B * L, D),
                   rtg.reshape(B * L, 16), rsg.reshape(B * L, 16))
    return (o1, o2)


# speedup vs baseline: 1.0293x; 1.0293x over previous
import functools
import jax
import jax.numpy as jnp
from jax import lax
from jax.experimental import pallas as pl
from jax.experimental.pallas import tpu as pltpu
from jax.experimental.pallas import tpu_sc as plsc

N = 8192
D = 64
HHR = 6
TOPK = 10
NEG = -3.0e38

BI = 256
RB = 64
BA = 512
LANE = 128

B = 1024
L = 16
NW = 32
IPW = (B // NW) * L
BT = 128


def _hidden_norm_body(x_ref, vs_ref, o_ref):
    h = x_ref[...]
    for i in range(HHR):
        v = vs_ref[0, i:i + 1, :]
        vdot = jnp.sum(v * v)
        w = lax.dot_general(h, v, (((1,), (1,)), ((), ())),
                            preferred_element_type=jnp.float32)
        h = h - lax.dot_general(w, v, (((1,), (0,)), ((), ())),
                                preferred_element_type=jnp.float32) / vdot
    norm = jnp.sqrt(jnp.sum(h * h, axis=1, keepdims=True))
    o_ref[...] = h / jnp.maximum(norm, 1e-12)


def _hidden_norm(x2, vs2):
    return pl.pallas_call(
        _hidden_norm_body,
        grid=(2 * N // BA,),
        in_specs=[
            pl.BlockSpec((BA, D), lambda i: (i, 0)),
            pl.BlockSpec((1, HHR, D), lambda i: (i // (N // BA), 0, 0)),
        ],
        out_specs=pl.BlockSpec((BA, D), lambda i: (i, 0)),
        out_shape=jax.ShapeDtypeStruct((2 * N, D), jnp.float32),
    )(x2, vs2)


def _topk_table_body(a_ref, b_ref, o_ref):
    s = lax.dot_general(a_ref[...], b_ref[...], (((1,), (1,)), ((), ())),
                        preferred_element_type=jnp.float32)
    sh = s.astype(jnp.bfloat16)
    parts = []
    for rb in range(BI // RB):
        t = [jnp.full((RB, LANE), NEG, dtype=jnp.bfloat16)
             for _ in range(TOPK)]
        for c in range(N // LANE):
            new = sh[rb * RB:(rb + 1) * RB, c * LANE:(c + 1) * LANE]
            for r in range(TOPK):
                hi = jnp.maximum(t[r], new)
                new = jnp.minimum(t[r], new)
                t[r] = hi
        lists = t
        w = LANE
        while w > 1:
            h = w // 2
            a = [x[:, :h] for x in lists]
            b = [x[:, h:w] for x in lists]
            nxt = []
            for k in range(TOPK):
                m = jnp.maximum(a[k], b[k])
                for i in range(k):
                    m = jnp.maximum(m, jnp.minimum(a[i], b[k - 1 - i]))
                nxt.append(m)
            lists = nxt
            w = h
        total = lists[0].astype(jnp.float32)
        for k in range(1, TOPK):
            total = total + lists[k].astype(jnp.float32)
        parts.append(total)
    total = jnp.concatenate(parts, axis=0)
    right = jnp.concatenate(
        [total * (1.0 / TOPK), jnp.zeros((BI, 63), jnp.float32)], axis=1)
    o_ref[...] = jnp.concatenate([a_ref[...], right], axis=1)


def _topk_table(hn):
    return pl.pallas_call(
        _topk_table_body,
        grid=(2 * N // BI,),
        in_specs=[
            pl.BlockSpec((BI, D), lambda i: (i, 0)),
            pl.BlockSpec((N, D), lambda i: (1 - i // (N // BI), 0)),
        ],
        out_specs=pl.BlockSpec((BI, 2 * D), lambda i: (i, 0)),
        out_shape=jax.ShapeDtypeStruct((2 * N, 2 * D), jnp.float32),
    )(hn, hn)


@functools.lru_cache(maxsize=1)
def _sc_gather_call():
    mesh = plsc.VectorSubcoreMesh(core_axis_name="c", subcore_axis_name="s",
                                  num_cores=2, num_subcores=16)

    @functools.partial(
        pl.kernel,
        out_type=[jax.ShapeDtypeStruct((NW, IPW, 2 * D), jnp.float32)] * 2,
        mesh=mesh,
        scratch_types=[
            pltpu.VMEM((IPW // 128, 128), jnp.int32),
            pltpu.VMEM((IPW // 128, 128), jnp.int32),
            pltpu.VMEM((IPW, 2 * D), jnp.float32),
            pltpu.SemaphoreType.DMA,
        ],
    )
    def sc_gather(tab_h, si_h, ti_h, so_h, to_h,
                  si_v, ti_v, rows_v, sem):
        wid = lax.axis_index("s") * 2 + lax.axis_index("c")
        pltpu.sync_copy(si_h.at[wid], si_v)
        pltpu.sync_copy(ti_h.at[wid], ti_v)
        for j in range(IPW // 128):
            pltpu.async_copy(tab_h.at[si_v.at[j]],
                             rows_v.at[pl.ds(j * 128, 128)], sem).wait()
        pltpu.sync_copy(rows_v, so_h.at[wid])
        for j in range(IPW // 128):
            pltpu.async_copy(tab_h.at[ti_v.at[j]],
                             rows_v.at[pl.ds(j * 128, 128)], sem).wait()
        pltpu.sync_copy(rows_v, to_h.at[wid])

    return sc_gather


def _tail_body(sg_ref, tg_ref, o1_ref, o2_ref):
    sl = sg_ref[...].reshape(BT, L, 2 * D)
    tl = tg_ref[...].reshape(BT, L, 2 * D)
    srow = sl[:, :, 0:D]
    trow = tl[:, :, 0:D]
    rtv = sl[:, :, D]
    rsv = tl[:, :, D]
    sr0 = sl[:, 0:1, 0:D]
    tr0 = tl[:, 0:1, 0:D]
    s2t = jnp.sum(sr0 * trow, axis=2)
    t2s = jnp.sum(tr0 * srow, axis=2)
    o1_ref[...] = 2.0 * s2t - rtv[:, 0:1] - rsv
    o2_ref[...] = 2.0 * t2s - rsv[:, 0:1] - rtv


def _tail(sg, tg):
    return pl.pallas_call(
        _tail_body,
        grid=(B // BT,),
        in_specs=[
            pl.BlockSpec((BT * L, 2 * D), lambda i: (i, 0)),
            pl.BlockSpec((BT * L, 2 * D), lambda i: (i, 0)),
        ],
        out_specs=[
            pl.BlockSpec((BT, L), lambda i: (i, 0)),
            pl.BlockSpec((BT, L), lambda i: (i, 0)),
        ],
        out_shape=[jax.ShapeDtypeStruct((B, L), jnp.float32)] * 2,
    )(sg, tg)


@jax.jit
def kernel(node_feat_src, node_feat_tgt, srcs_index, tgts_index, src_vs, tgt_vs):
    x2 = jnp.concatenate([node_feat_src, node_feat_tgt], axis=0)
    vs2 = jnp.concatenate([src_vs.reshape(1, HHR, D),
                           tgt_vs.reshape(1, HHR, D)], axis=0)
    hn = _hidden_norm(x2, vs2)

    ctab = _topk_table(hn)

    si3 = srcs_index.astype(jnp.int32).reshape(NW, IPW // 128, 128)
    ti3 = (tgts_index.astype(jnp.int32) + N).reshape(NW, IPW // 128, 128)
    sg, tg = _sc_gather_call()(ctab, si3, ti3)

    o1, o2 = _tail(sg.reshape(B * L, 2 * D), tg.reshape(B * L, 2 * D))
    return (o1, o2)


# speedup vs baseline: 1.0384x; 1.0089x over previous
import functools
import jax
import jax.numpy as jnp
from jax import lax
from jax.experimental import pallas as pl
from jax.experimental.pallas import tpu as pltpu
from jax.experimental.pallas import tpu_sc as plsc

N = 8192
D = 64
HHR = 6
TOPK = 10
NEG = -3.0e38

BI = 256
RB = 64
BA = 512
LANE = 128

B = 1024
L = 16
NW = 32
IPW = (B // NW) * L
BT = 128


def _hidden_norm_body(x_ref, vs_ref, o_ref):
    h = x_ref[...]
    for i in range(HHR):
        v = vs_ref[i:i + 1, :]
        vdot = jnp.sum(v * v)
        w = lax.dot_general(h, v, (((1,), (1,)), ((), ())),
                            preferred_element_type=jnp.float32)
        h = h - lax.dot_general(w, v, (((1,), (0,)), ((), ())),
                                preferred_element_type=jnp.float32) / vdot
    norm = jnp.sqrt(jnp.sum(h * h, axis=1, keepdims=True))
    o_ref[...] = h / jnp.maximum(norm, 1e-12)


def _hidden_norm(x, vs):
    return pl.pallas_call(
        _hidden_norm_body,
        grid=(N // BA,),
        in_specs=[
            pl.BlockSpec((BA, D), lambda i: (i, 0)),
            pl.BlockSpec((HHR, D), lambda i: (0, 0)),
        ],
        out_specs=pl.BlockSpec((BA, D), lambda i: (i, 0)),
        out_shape=jax.ShapeDtypeStruct((N, D), jnp.float32),
    )(x, vs)


def _topk_table_body(a_ref, b_ref, o_ref):
    s = lax.dot_general(a_ref[...], b_ref[...], (((1,), (1,)), ((), ())),
                        preferred_element_type=jnp.float32)
    sh = s.astype(jnp.bfloat16)
    parts = []
    for rb in range(BI // RB):
        t = [jnp.full((RB, LANE), NEG, dtype=jnp.bfloat16)
             for _ in range(TOPK)]
        for c in range(N // LANE):
            new = sh[rb * RB:(rb + 1) * RB, c * LANE:(c + 1) * LANE]
            for r in range(TOPK):
                hi = jnp.maximum(t[r], new)
                new = jnp.minimum(t[r], new)
                t[r] = hi
        lists = t
        w = LANE
        while w > 1:
            h = w // 2
            a = [x[:, :h] for x in lists]
            b = [x[:, h:w] for x in lists]
            nxt = []
            for k in range(TOPK):
                m = jnp.maximum(a[k], b[k])
                for i in range(k):
                    m = jnp.maximum(m, jnp.minimum(a[i], b[k - 1 - i]))
                nxt.append(m)
            lists = nxt
            w = h
        total = lists[0].astype(jnp.float32)
        for k in range(1, TOPK):
            total = total + lists[k].astype(jnp.float32)
        parts.append(total)
    total = jnp.concatenate(parts, axis=0)
    right = jnp.concatenate(
        [total * (1.0 / TOPK), jnp.zeros((BI, 63), jnp.float32)], axis=1)
    o_ref[...] = jnp.concatenate([a_ref[...], right], axis=1)


def _topk_table(a, b):
    return pl.pallas_call(
        _topk_table_body,
        grid=(N // BI,),
        in_specs=[
            pl.BlockSpec((BI, D), lambda i: (i, 0)),
            pl.BlockSpec((N, D), lambda i: (0, 0)),
        ],
        out_specs=pl.BlockSpec((BI, 2 * D), lambda i: (i, 0)),
        out_shape=jax.ShapeDtypeStruct((N, 2 * D), jnp.float32),
    )(a, b)


@functools.lru_cache(maxsize=1)
def _sc_gather_call():
    mesh = plsc.VectorSubcoreMesh(core_axis_name="c", subcore_axis_name="s",
                                  num_cores=2, num_subcores=16)

    @functools.partial(
        pl.kernel,
        out_type=[jax.ShapeDtypeStruct((NW, IPW, 2 * D), jnp.float32)] * 2,
        mesh=mesh,
        scratch_types=[
            pltpu.VMEM((IPW // 128, 128), jnp.int32),
            pltpu.VMEM((IPW // 128, 128), jnp.int32),
            pltpu.VMEM((IPW, 2 * D), jnp.float32),
            pltpu.SemaphoreType.DMA,
        ],
    )
    def sc_gather(stab_h, ttab_h, si_h, ti_h, so_h, to_h,
                  si_v, ti_v, rows_v, sem):
        wid = lax.axis_index("s") * 2 + lax.axis_index("c")
        pltpu.sync_copy(si_h.at[wid], si_v)
        pltpu.sync_copy(ti_h.at[wid], ti_v)
        for j in range(IPW // 128):
            pltpu.async_copy(stab_h.at[si_v.at[j]],
                             rows_v.at[pl.ds(j * 128, 128)], sem).wait()
        pltpu.sync_copy(rows_v, so_h.at[wid])
        for j in range(IPW // 128):
            pltpu.async_copy(ttab_h.at[ti_v.at[j]],
                             rows_v.at[pl.ds(j * 128, 128)], sem).wait()
        pltpu.sync_copy(rows_v, to_h.at[wid])

    return sc_gather


def _tail_body(sg_ref, tg_ref, o1_ref, o2_ref):
    sl = sg_ref[...].reshape(BT, L, 2 * D)
    tl = tg_ref[...].reshape(BT, L, 2 * D)
    srow = sl[:, :, 0:D]
    trow = tl[:, :, 0:D]
    rtv = sl[:, :, D]
    rsv = tl[:, :, D]
    sr0 = sl[:, 0:1, 0:D]
    tr0 = tl[:, 0:1, 0:D]
    s2t = jnp.sum(sr0 * trow, axis=2)
    t2s = jnp.sum(tr0 * srow, axis=2)
    o1_ref[...] = 2.0 * s2t - rtv[:, 0:1] - rsv
    o2_ref[...] = 2.0 * t2s - rsv[:, 0:1] - rtv


def _tail(sg, tg):
    return pl.pallas_call(
        _tail_body,
        grid=(B // BT,),
        in_specs=[
            pl.BlockSpec((BT * L, 2 * D), lambda i: (i, 0)),
            pl.BlockSpec((BT * L, 2 * D), lambda i: (i, 0)),
        ],
        out_specs=[
            pl.BlockSpec((BT, L), lambda i: (i, 0)),
            pl.BlockSpec((BT, L), lambda i: (i, 0)),
        ],
        out_shape=[jax.ShapeDtypeStruct((B, L), jnp.float32)] * 2,
    )(sg, tg)


@jax.jit
def kernel(node_feat_src, node_feat_tgt, srcs_index, tgts_index, src_vs, tgt_vs):
    src_vs2 = src_vs.reshape(HHR, D)
    tgt_vs2 = tgt_vs.reshape(HHR, D)
    src_n = _hidden_norm(node_feat_src, src_vs2)
    tgt_n = _hidden_norm(node_feat_tgt, tgt_vs2)

    stab = _topk_table(src_n, tgt_n)
    ttab = _topk_table(tgt_n, src_n)

    si3 = srcs_index.astype(jnp.int32).reshape(NW, IPW // 128, 128)
    ti3 = tgts_index.astype(jnp.int32).reshape(NW, IPW // 128, 128)
    sg, tg = _sc_gather_call()(stab, ttab, si3, ti3)

    o1, o2 = _tail(sg.reshape(B * L, 2 * D), tg.reshape(B * L, 2 * D))
    return (o1, o2)
